# baseline (speedup 1.0000x reference)
import functools

import jax
import jax.numpy as jnp
from jax.experimental import pallas as pl

N = 10000
D = 256
ROWS = 400


def _proj_body(x_ref, wt_ref, b_ref, o_ref):
    acc = jnp.dot(x_ref[...], wt_ref[...], preferred_element_type=jnp.float32)
    o_ref[...] = jnp.maximum(acc + b_ref[...], 0.0)


def _proj(x, wt, b):
    grid = (x.shape[0] // ROWS,)
    return pl.pallas_call(
        _proj_body,
        grid=grid,
        in_specs=[
            pl.BlockSpec((ROWS, D), lambda i: (i, 0)),
            pl.BlockSpec((D, D), lambda i: (0, 0)),
            pl.BlockSpec((1, D), lambda i: (0, 0)),
        ],
        out_specs=pl.BlockSpec((ROWS, D), lambda i: (i, 0)),
        out_shape=jax.ShapeDtypeStruct((x.shape[0], D), jnp.float32),
    )(x, wt, b)


def _out_body(aggr_ref, wlt_ref, bl_ref, x_ref, wrt_ref, o_ref, *, do_relu):
    acc = jnp.dot(aggr_ref[...], wlt_ref[...], preferred_element_type=jnp.float32)
    acc = acc + bl_ref[...]
    acc = acc + jnp.dot(x_ref[...], wrt_ref[...], preferred_element_type=jnp.float32)
    nrm = jnp.sqrt(jnp.sum(acc * acc, axis=-1, keepdims=True))
    res = acc / jnp.maximum(nrm, 1e-12)
    if do_relu:
        res = jnp.maximum(res, 0.0)
    o_ref[...] = res


def _out(aggr, wlt, bl, x, wrt, do_relu):
    grid = (x.shape[0] // ROWS,)
    return pl.pallas_call(
        functools.partial(_out_body, do_relu=do_relu),
        grid=grid,
        in_specs=[
            pl.BlockSpec((ROWS, D), lambda i: (i, 0)),
            pl.BlockSpec((D, D), lambda i: (0, 0)),
            pl.BlockSpec((1, D), lambda i: (0, 0)),
            pl.BlockSpec((ROWS, D), lambda i: (i, 0)),
            pl.BlockSpec((D, D), lambda i: (0, 0)),
        ],
        out_specs=pl.BlockSpec((ROWS, D), lambda i: (i, 0)),
        out_shape=jax.ShapeDtypeStruct((x.shape[0], D), jnp.float32),
    )(aggr, wlt, bl, x, wrt)


def _segment_max(xp, src, dst):
    msg = xp[src]
    return jax.ops.segment_max(msg, dst, num_segments=N, indices_are_sorted=False)


def _layer(x, edge_index, Wp, bp, Wl, bl, Wr, do_relu):
    src = edge_index[0]
    dst = edge_index[1]
    xp = _proj(x, Wp.T, bp.reshape(1, D))
    aggr = _segment_max(xp, src, dst)
    aggr = jnp.where(jnp.isneginf(aggr), 0.0, aggr)
    return _out(aggr, Wl.T, bl.reshape(1, D), x, Wr.T, do_relu)


@jax.jit
def kernel(x, edge_index, Wp1, bp1, Wl1, bl1, Wr1, Wp2, bp2, Wl2, bl2, Wr2):
    h = _layer(x, edge_index, Wp1, bp1, Wl1, bl1, Wr1, True)
    return _layer(h, edge_index, Wp2, bp2, Wl2, bl2, Wr2, False)
